# baseline (speedup 1.0000x reference)
import functools

import jax
import jax.numpy as jnp
import numpy as np
from jax import lax
from jax.experimental import pallas as pl
from jax.experimental.pallas import tpu as pltpu
from jax.experimental.pallas import tpu_sc as plsc

_N = 10000
_E = 320000
_NP = 10240
_RB = 2048
_K = 64
_HID = 256
_EMB = 128
_NEG_INF = -1e9
_NQ = 256
_PREC = lax.Precision.HIGHEST


def _kin_body(x_ref, w_ref, b_ref, o_ref):
    o_ref[...] = jax.nn.relu(
        jnp.dot(x_ref[...], w_ref[...], precision=_PREC) + b_ref[...])


def _k_in(x_p, w, b2):
    return pl.pallas_call(
        _kin_body,
        grid=(_NP // _RB,),
        in_specs=[pl.BlockSpec((_RB, _EMB), lambda i: (i, 0)),
                  pl.BlockSpec((_EMB, _HID), lambda i: (0, 0)),
                  pl.BlockSpec((1, _HID), lambda i: (0, 0))],
        out_specs=pl.BlockSpec((_RB, _HID), lambda i: (i, 0)),
        out_shape=jax.ShapeDtypeStruct((_NP, _HID), jnp.float32),
    )(x_p, w, b2)


def _kdense_body(h_ref, w_ref, ws_ref, wd_ref, tab_ref, as_ref, ad_ref):
    h = h_ref[...]
    tab_ref[...] = jnp.dot(h, w_ref[0], precision=_PREC)
    as_ref[...] = jnp.dot(h, ws_ref[0], precision=_PREC)
    ad_ref[...] = jnp.dot(h, wd_ref[0], precision=_PREC)


def _k_dense(h, wt, wsb, wdb, fh):
    nb = _NP // _RB
    return pl.pallas_call(
        _kdense_body,
        grid=(2, nb),
        in_specs=[pl.BlockSpec((_RB, _HID), lambda j, i: (i, 0)),
                  pl.BlockSpec((1, _HID, fh), lambda j, i: (j, 0, 0)),
                  pl.BlockSpec((1, _HID, 16), lambda j, i: (j, 0, 0)),
                  pl.BlockSpec((1, _HID, 16), lambda j, i: (j, 0, 0))],
        out_specs=[pl.BlockSpec((_RB, fh), lambda j, i: (j * nb + i, 0)),
                   pl.BlockSpec((_RB, 16), lambda j, i: (j * nb + i, 0)),
                   pl.BlockSpec((_RB, 16), lambda j, i: (j * nb + i, 0))],
        out_shape=(jax.ShapeDtypeStruct((2 * _NP, fh), jnp.float32),
                   jax.ShapeDtypeStruct((2 * _NP, 16), jnp.float32),
                   jax.ShapeDtypeStruct((2 * _NP, 16), jnp.float32)),
    )(h, wt, wsb, wdb)


def _norm_block(a, b, da, db, ea, bias, g, beta):
    div_a = jnp.dot(1.0 / (da + 1e-16), ea, precision=_PREC)
    div_b = jnp.dot(1.0 / (db + 1e-16), ea, precision=_PREC)
    v = jnp.concatenate([a * div_a, b * div_b], axis=1) + bias
    mu = jnp.mean(v, axis=1, keepdims=True)
    var = jnp.mean((v - mu) ** 2, axis=1, keepdims=True)
    return (v - mu) / jnp.sqrt(var + 1e-5) * g + beta


def _knorm_body(a_ref, b_ref, da_ref, db_ref, ea_ref, bias_ref, g_ref,
                beta_ref, o_ref):
    v = _norm_block(a_ref[0], b_ref[0], da_ref[0], db_ref[0], ea_ref[...],
                    bias_ref[...], g_ref[...], beta_ref[...])
    o_ref[...] = jax.nn.relu(v)


def _k_norm(acc, den, ea, bias, g, beta, fh, fo):
    return pl.pallas_call(
        _knorm_body,
        grid=(_NP // _RB,),
        in_specs=[pl.BlockSpec((1, _RB, fh), lambda i: (0, i, 0)),
                  pl.BlockSpec((1, _RB, fh), lambda i: (1, i, 0)),
                  pl.BlockSpec((1, _RB, 16), lambda i: (0, i, 0)),
                  pl.BlockSpec((1, _RB, 16), lambda i: (1, i, 0)),
                  pl.BlockSpec((16, fh), lambda i: (0, 0)),
                  pl.BlockSpec((1, fo), lambda i: (0, 0)),
                  pl.BlockSpec((1, fo), lambda i: (0, 0)),
                  pl.BlockSpec((1, fo), lambda i: (0, 0))],
        out_specs=pl.BlockSpec((_RB, fo), lambda i: (i, 0)),
        out_shape=jax.ShapeDtypeStruct((_NP, fo), jnp.float32),
    )(acc, acc, den, den, ea, bias, g, beta)


def _kemb_body(a_ref, b_ref, da_ref, db_ref, ea_ref, bias_ref, g_ref,
               beta_ref, h0_ref, sw_ref, sb_ref, o_ref):
    v = _norm_block(a_ref[0], b_ref[0], da_ref[0], db_ref[0], ea_ref[...],
                    bias_ref[...], g_ref[...], beta_ref[...])
    o_ref[...] = v + jnp.dot(h0_ref[...], sw_ref[...],
                             precision=_PREC) + sb_ref[...]


def _k_emb(acc, den, ea, bias, g, beta, h0, sw, sb, fh, fo):
    return pl.pallas_call(
        _kemb_body,
        grid=(_NP // _RB,),
        in_specs=[pl.BlockSpec((1, _RB, fh), lambda i: (0, i, 0)),
                  pl.BlockSpec((1, _RB, fh), lambda i: (1, i, 0)),
                  pl.BlockSpec((1, _RB, 16), lambda i: (0, i, 0)),
                  pl.BlockSpec((1, _RB, 16), lambda i: (1, i, 0)),
                  pl.BlockSpec((16, fh), lambda i: (0, 0)),
                  pl.BlockSpec((1, fo), lambda i: (0, 0)),
                  pl.BlockSpec((1, fo), lambda i: (0, 0)),
                  pl.BlockSpec((1, fo), lambda i: (0, 0)),
                  pl.BlockSpec((_RB, _HID), lambda i: (i, 0)),
                  pl.BlockSpec((_HID, fo), lambda i: (0, 0)),
                  pl.BlockSpec((1, fo), lambda i: (0, 0))],
        out_specs=pl.BlockSpec((_RB, fo), lambda i: (i, 0)),
        out_shape=jax.ShapeDtypeStruct((_NP, fo), jnp.float32),
    )(acc, acc, den, den, ea, bias, g, beta, h0, sw, sb)


def _kq_body(e_ref, fw1, fb1, fw2, fb2, mw1, mb1, mw2, mb2, bw,
             qf_ref, qm_ref):
    q = e_ref[...]
    f = jax.nn.relu(jnp.dot(q, fw1[...], precision=_PREC) + fb1[...])
    f = jax.nn.relu(jnp.dot(f, fw2[...], precision=_PREC) + fb2[...])
    qf_ref[...] = jnp.dot(f, bw[...], precision=_PREC)
    m = jax.nn.relu(jnp.dot(q, mw1[...], precision=_PREC) + mb1[...])
    m = jax.nn.relu(jnp.dot(m, mw2[...], precision=_PREC) + mb2[...])
    qm_ref[...] = jnp.dot(m, bw[...], precision=_PREC)


def _k_q(emb, fw1, fb1, fw2, fb2, mw1, mb1, mw2, mb2, bw, nq):
    return pl.pallas_call(
        _kq_body,
        in_specs=[pl.BlockSpec((nq, _EMB), lambda: (0, 0)),
                  pl.BlockSpec((_EMB, _HID), lambda: (0, 0)),
                  pl.BlockSpec((1, _HID), lambda: (0, 0)),
                  pl.BlockSpec((_HID, _HID), lambda: (0, 0)),
                  pl.BlockSpec((1, _HID), lambda: (0, 0)),
                  pl.BlockSpec((_EMB, _HID), lambda: (0, 0)),
                  pl.BlockSpec((1, _HID), lambda: (0, 0)),
                  pl.BlockSpec((_HID, _HID), lambda: (0, 0)),
                  pl.BlockSpec((1, _HID), lambda: (0, 0)),
                  pl.BlockSpec((_HID, _EMB), lambda: (0, 0))],
        out_specs=[pl.BlockSpec((nq, _EMB), lambda: (0, 0)),
                   pl.BlockSpec((nq, _EMB), lambda: (0, 0))],
        out_shape=(jax.ShapeDtypeStruct((nq, _EMB), jnp.float32),
                   jax.ShapeDtypeStruct((nq, _EMB), jnp.float32)),
    )(emb, fw1, fb1, fw2, fb2, mw1, mb1, mw2, mb2, bw)


def _kscore_body(qf_ref, qm_ref, e_ref, gen_ref, gd_ref, genq_ref, bb_ref,
                 fo_ref, mo_ref):
    emb = e_ref[...]
    dn = (((1,), (1,)), ((), ()))
    sf = lax.dot_general(qf_ref[...], emb, dn, precision=_PREC) + bb_ref[...]
    sm = lax.dot_general(qm_ref[...], emb, dn, precision=_PREC) + bb_ref[...]
    gd = gen_ref[0:1, :] - genq_ref[...]
    invalid = (gd < 0.5) | (gd > 2.0)
    male = gd_ref[0:1, :] == 1.0
    female = gd_ref[0:1, :] == 0.0
    fo_ref[...] = jnp.where(male & (~invalid), sf, _NEG_INF)
    mo_ref[...] = jnp.where(female & (~invalid), sm, _NEG_INF)


def _k_score(qf, qm, emb, genr8, gender8, genq, bb, nq):
    nb = _NP // _RB
    return pl.pallas_call(
        _kscore_body,
        grid=(nb,),
        in_specs=[pl.BlockSpec((nq, _EMB), lambda i: (0, 0)),
                  pl.BlockSpec((nq, _EMB), lambda i: (0, 0)),
                  pl.BlockSpec((_RB, _EMB), lambda i: (i, 0)),
                  pl.BlockSpec((8, _RB), lambda i: (0, i)),
                  pl.BlockSpec((8, _RB), lambda i: (0, i)),
                  pl.BlockSpec((nq, 1), lambda i: (0, 0)),
                  pl.BlockSpec((1, 1), lambda i: (0, 0))],
        out_specs=[pl.BlockSpec((nq, _RB), lambda i: (0, i)),
                   pl.BlockSpec((nq, _RB), lambda i: (0, i))],
        out_shape=(jax.ShapeDtypeStruct((nq, _NP), jnp.float32),
                   jax.ShapeDtypeStruct((nq, _NP), jnp.float32)),
    )(qf, qm, emb, genr8, gender8, genq, bb)


def _edge_chunks():
    npt = -(-_E // (16 * _K))
    npt = ((npt + 2) // 3) * 3
    return npt


def _sc_edge(tab, asb, adb, srcoff, dstoff, dst3, fh, heads):
    npt = _edge_chunks()
    nrt = _NP // 16
    nzc = nrt // _K + (1 if nrt % _K else 0)
    chw = min(_HID // heads, fh)
    ch = fh // chw
    mesh = plsc.VectorSubcoreMesh(core_axis_name="c", subcore_axis_name="s",
                                  num_cores=2, num_subcores=16)

    @functools.partial(
        pl.kernel,
        out_type=(jax.ShapeDtypeStruct((2, _NP, fh), jnp.float32),
                  jax.ShapeDtypeStruct((2, _NP, 16), jnp.float32)),
        mesh=mesh,
        compiler_params=pltpu.CompilerParams(use_tc_tiling_on_sc=False),
        scratch_types=[
            [pltpu.VMEM((_K,), jnp.int32)] * 3,
            [pltpu.VMEM((_K,), jnp.int32)] * 3,
            [pltpu.VMEM((_K,), jnp.int32)] * 3,
            [pltpu.VMEM((_K, 16), jnp.float32)] * 3,
            [pltpu.VMEM((_K, 16), jnp.float32)] * 3,
            [pltpu.VMEM((_K, 16), jnp.float32)] * 3,
            [pltpu.VMEM((_K, fh), jnp.float32)] * 3,
            pltpu.VMEM_SHARED((_NP, fh), jnp.float32),
            pltpu.VMEM_SHARED((_NP, 16), jnp.float32),
            [pltpu.SemaphoreType.DMA] * 3,
            [pltpu.SemaphoreType.DMA] * 3,
            [pltpu.SemaphoreType.DMA] * 3,
        ])
    def k(tab_h, as_h, ad_h, so_h, do_h, dr_h, acc_o, den_o, s2, d2, dr,
          av, bv, wv, rows, acc_sh, den_sh, si, sg, ss):
        c = lax.axis_index("c")
        s = lax.axis_index("s")
        zero16 = jnp.zeros((16,), jnp.float32)

        def zb(i, carry):
            for v in range(fh // 16):
                rows[0][i, pl.ds(16 * v, 16)] = zero16
            wv[0][i, :] = zero16
            return carry
        lax.fori_loop(0, _K, zb, 0)

        for q in range(nzc):
            base = s * nrt + min(q * _K, nrt - _K)
            pltpu.sync_copy(rows[0], acc_sh.at[pl.ds(base, _K)])
            pltpu.sync_copy(wv[0], den_sh.at[pl.ds(base, _K)])
        plsc.subcore_barrier()

        def i_copies(n, b):
            return (pltpu.make_async_copy(so_h.at[c, s, n], s2[b], si[b]),
                    pltpu.make_async_copy(do_h.at[c, s, n], d2[b], si[b]),
                    pltpu.make_async_copy(dr_h.at[s, n], dr[b], si[b]))

        def g_copies(n, b):
            return (pltpu.make_async_copy(as_h.at[s2[b]], av[b], sg[b]),
                    pltpu.make_async_copy(ad_h.at[d2[b]], bv[b], sg[b]),
                    pltpu.make_async_copy(tab_h.at[s2[b]], rows[b], sg[b]))

        def s_copies(n, b):
            return (pltpu.make_async_copy(wv[b], den_sh.at[dr[b]], ss[b]),
                    pltpu.make_async_copy(rows[b], acc_sh.at[dr[b]], ss[b]))

        def piece(n, b):
            bn = (b + 1) % 3
            bnn = (b + 2) % 3

            @pl.when(n >= 2)
            def _():
                for cp in s_copies(n - 2, bn):
                    cp.wait()

            @pl.when(n + 1 < npt)
            def _():
                for cp in i_copies(n + 1, bn):
                    cp.wait()
                for cp in g_copies(n + 1, bn):
                    cp.start()

            @pl.when(n + 2 < npt)
            def _():
                for cp in i_copies(n + 2, bnn):
                    cp.start()

            for cp in g_copies(n, b):
                cp.wait()

            def edge(k2, cy):
                t = av[b][k2, :] + bv[b][k2, :]
                t = jnp.maximum(t, 0.0) + 0.2 * jnp.minimum(t, 0.0)
                w = jnp.exp(t)
                wv[b][k2, :] = w
                for hh in range(ch):
                    wsc = w[hh]
                    for v2 in range(chw // 16):
                        sl = pl.ds(hh * chw + 16 * v2, 16)
                        rows[b][k2, sl] = rows[b][k2, sl] * wsc
                return cy
            lax.fori_loop(0, _K, edge, 0, unroll=2)

            pltpu.async_copy(wv[b], den_sh.at[dr[b]], ss[b], add=True)
            pltpu.async_copy(rows[b], acc_sh.at[dr[b]], ss[b], add=True)

        for cp in i_copies(0, 0):
            cp.start()
        for cp in i_copies(1, 1):
            cp.start()
        for cp in i_copies(0, 0):
            cp.wait()
        for cp in g_copies(0, 0):
            cp.start()

        def chunk3(t, carry):
            piece(3 * t, 0)
            piece(3 * t + 1, 1)
            piece(3 * t + 2, 2)
            return carry
        lax.fori_loop(0, npt // 3, chunk3, 0)

        for cp in s_copies(npt - 2, (npt - 2) % 3):
            cp.wait()
        for cp in s_copies(npt - 1, (npt - 1) % 3):
            cp.wait()
        plsc.subcore_barrier()
        rbase = s * nrt
        pltpu.sync_copy(acc_sh.at[pl.ds(rbase, nrt)],
                        acc_o.at[c, pl.ds(rbase, nrt)])
        pltpu.sync_copy(den_sh.at[pl.ds(rbase, nrt)],
                        den_o.at[c, pl.ds(rbase, nrt)])

    return k(tab, asb, adb, srcoff, dstoff, dst3)


def _expand_mats(heads, oph, fh):
    ea = np.zeros((16, fh), np.float32)
    chw = min(oph, fh)
    for h in range(fh // chw):
        ea[h, h * chw:(h + 1) * chw] = 1.0
    return jnp.asarray(ea)


def _pad16(v):
    return jnp.pad(v, ((0, 0), (0, 16 - v.shape[1])))


def kernel(x, edge_index, test_mask, gender, generation, W_in, b_in,
           gat_W_0, gat_as_0, gat_ad_0, gat_b_0, ln_g_0, ln_b_0,
           gat_W_1, gat_as_1, gat_ad_1, gat_b_1, ln_g_1, ln_b_1,
           gat_W_2, gat_as_2, gat_ad_2, gat_b_2, ln_g_2, ln_b_2,
           gat_W_3, gat_as_3, gat_ad_3, gat_b_3, ln_g_3, ln_b_3,
           skip_W, skip_b, fa_W1, fa_b1, fa_W2, fa_b2,
           mo_W1, mo_b1, mo_W2, mo_b2, bil_W, bil_b):
    f32 = jnp.float32
    nq = _NQ
    x_p = jnp.pad(x.astype(f32), ((0, _NP - _N), (0, 0)))
    src = edge_index[0].astype(jnp.int32)
    dst = edge_index[1].astype(jnp.int32)
    npt = _edge_chunks()
    ep = 16 * _K * npt
    srcp = jnp.concatenate([src, jnp.full((ep - _E,), _N, jnp.int32)])
    dstp = jnp.concatenate([dst, jnp.full((ep - _E,), _N, jnp.int32)])
    srcoff = jnp.stack([srcp, srcp + _NP]).reshape(2, 16, npt, _K)
    dstoff = jnp.stack([dstp, dstp + _NP]).reshape(2, 16, npt, _K)
    dst3 = dstp.reshape(16, npt, _K)

    h = _k_in(x_p, W_in.astype(f32), b_in.astype(f32).reshape(1, -1))
    h0 = h

    gat_W = [gat_W_0, gat_W_1, gat_W_2, gat_W_3]
    gat_as = [gat_as_0, gat_as_1, gat_as_2, gat_as_3]
    gat_ad = [gat_ad_0, gat_ad_1, gat_ad_2, gat_ad_3]
    gat_b = [gat_b_0, gat_b_1, gat_b_2, gat_b_3]
    ln_g = [ln_g_0, ln_g_1, ln_g_2, ln_g_3]
    ln_b = [ln_b_0, ln_b_1, ln_b_2, ln_b_3]

    emb = None
    for i in range(4):
        heads, oph = (8, _HID // 8) if i < 3 else (1, _EMB)
        fo = heads * oph
        fh = fo // 2
        w = gat_W[i].astype(f32)
        wt = jnp.stack([w[:, :fh], w[:, fh:]], 0)
        ws = jnp.einsum('fhd,hd->fh', w.reshape(_HID, heads, oph),
                        gat_as[i].astype(f32))
        wd = jnp.einsum('fhd,hd->fh', w.reshape(_HID, heads, oph),
                        gat_ad[i].astype(f32))
        nlh = heads // 2
        sbanks, dbanks = [], []
        for cc in range(2):
            hs = cc * nlh
            ncol = max(nlh, 1)
            sbanks.append(_pad16(ws[:, hs:hs + ncol]))
            dbanks.append(_pad16(wd[:, hs:hs + ncol]))
        wsb = jnp.stack(sbanks, 0)
        wdb = jnp.stack(dbanks, 0)
        tab, asb, adb = _k_dense(h, wt, wsb, wdb, fh)
        acc, den = _sc_edge(tab, asb, adb, srcoff, dstoff, dst3, fh, heads)
        ea = _expand_mats(heads, oph, fh)
        bias = gat_b[i].astype(f32).reshape(1, -1)
        g = ln_g[i].astype(f32).reshape(1, -1)
        beta = ln_b[i].astype(f32).reshape(1, -1)
        if i < 3:
            h = _k_norm(acc, den, ea, bias, g, beta, fh, fo)
        else:
            emb = _k_emb(acc, den, ea, bias, g, beta, h0,
                         skip_W.astype(f32), skip_b.astype(f32).reshape(1, -1),
                         fh, fo)

    qf, qm = _k_q(emb[:nq], fa_W1.astype(f32), fa_b1.astype(f32).reshape(1, -1),
                  fa_W2.astype(f32), fa_b2.astype(f32).reshape(1, -1),
                  mo_W1.astype(f32), mo_b1.astype(f32).reshape(1, -1),
                  mo_W2.astype(f32), mo_b2.astype(f32).reshape(1, -1),
                  bil_W.astype(f32), nq)

    gender_p = jnp.pad(gender.astype(f32), (0, _NP - _N), constant_values=5.0)
    genr_p = jnp.pad(generation.astype(f32), (0, _NP - _N))
    gender8 = jnp.broadcast_to(gender_p[None, :], (8, _NP))
    genr8 = jnp.broadcast_to(genr_p[None, :], (8, _NP))
    genq = generation.astype(f32)[:nq].reshape(nq, 1)
    bb = bil_b.astype(f32).reshape(1, 1)

    father, mother = _k_score(qf, qm, emb, genr8, gender8, genq, bb, nq)
    return father[:, :_N], mother[:, :_N], emb[:_N]


# speedup vs baseline: 1.0962x; 1.0962x over previous
import functools

import jax
import jax.numpy as jnp
import numpy as np
from jax import lax
from jax.experimental import pallas as pl
from jax.experimental.pallas import tpu as pltpu
from jax.experimental.pallas import tpu_sc as plsc

_N = 10000
_E = 320000
_NP = 10240
_RB = 2048
_K = 48
_HID = 256
_EMB = 128
_NEG_INF = -1e9
_NQ = 256
_PREC = lax.Precision.HIGHEST


def _kin_body(x_ref, w_ref, b_ref, o_ref):
    o_ref[...] = jax.nn.relu(
        jnp.dot(x_ref[...], w_ref[...], precision=_PREC) + b_ref[...])


def _k_in(x_p, w, b2):
    return pl.pallas_call(
        _kin_body,
        grid=(_NP // _RB,),
        in_specs=[pl.BlockSpec((_RB, _EMB), lambda i: (i, 0)),
                  pl.BlockSpec((_EMB, _HID), lambda i: (0, 0)),
                  pl.BlockSpec((1, _HID), lambda i: (0, 0))],
        out_specs=pl.BlockSpec((_RB, _HID), lambda i: (i, 0)),
        out_shape=jax.ShapeDtypeStruct((_NP, _HID), jnp.float32),
    )(x_p, w, b2)


def _kdense_body(h_ref, w_ref, ws_ref, wd_ref, tab_ref, as_ref, ad_ref):
    h = h_ref[...]
    tab_ref[...] = jnp.dot(h, w_ref[0], precision=_PREC)
    as_ref[...] = jnp.dot(h, ws_ref[0], precision=_PREC)
    ad_ref[...] = jnp.dot(h, wd_ref[0], precision=_PREC)


def _k_dense(h, wt, wsb, wdb, fh):
    nb = _NP // _RB
    return pl.pallas_call(
        _kdense_body,
        grid=(2, nb),
        in_specs=[pl.BlockSpec((_RB, _HID), lambda j, i: (i, 0)),
                  pl.BlockSpec((1, _HID, fh), lambda j, i: (j, 0, 0)),
                  pl.BlockSpec((1, _HID, 16), lambda j, i: (j, 0, 0)),
                  pl.BlockSpec((1, _HID, 16), lambda j, i: (j, 0, 0))],
        out_specs=[pl.BlockSpec((_RB, fh), lambda j, i: (j * nb + i, 0)),
                   pl.BlockSpec((_RB, 16), lambda j, i: (j * nb + i, 0)),
                   pl.BlockSpec((_RB, 16), lambda j, i: (j * nb + i, 0))],
        out_shape=(jax.ShapeDtypeStruct((2 * _NP, fh), jnp.float32),
                   jax.ShapeDtypeStruct((2 * _NP, 16), jnp.float32),
                   jax.ShapeDtypeStruct((2 * _NP, 16), jnp.float32)),
    )(h, wt, wsb, wdb)


def _norm_block(a, b, da, db, ea, bias, g, beta):
    div_a = jnp.dot(1.0 / (da + 1e-16), ea, precision=_PREC)
    div_b = jnp.dot(1.0 / (db + 1e-16), ea, precision=_PREC)
    v = jnp.concatenate([a * div_a, b * div_b], axis=1) + bias
    mu = jnp.mean(v, axis=1, keepdims=True)
    var = jnp.mean((v - mu) ** 2, axis=1, keepdims=True)
    return (v - mu) / jnp.sqrt(var + 1e-5) * g + beta


def _knorm_body(a_ref, b_ref, da_ref, db_ref, ea_ref, bias_ref, g_ref,
                beta_ref, o_ref):
    v = _norm_block(a_ref[0], b_ref[0], da_ref[0], db_ref[0], ea_ref[...],
                    bias_ref[...], g_ref[...], beta_ref[...])
    o_ref[...] = jax.nn.relu(v)


def _k_norm(acc, den, ea, bias, g, beta, fh, fo):
    return pl.pallas_call(
        _knorm_body,
        grid=(_NP // _RB,),
        in_specs=[pl.BlockSpec((1, _RB, fh), lambda i: (0, i, 0)),
                  pl.BlockSpec((1, _RB, fh), lambda i: (1, i, 0)),
                  pl.BlockSpec((1, _RB, 16), lambda i: (0, i, 0)),
                  pl.BlockSpec((1, _RB, 16), lambda i: (1, i, 0)),
                  pl.BlockSpec((16, fh), lambda i: (0, 0)),
                  pl.BlockSpec((1, fo), lambda i: (0, 0)),
                  pl.BlockSpec((1, fo), lambda i: (0, 0)),
                  pl.BlockSpec((1, fo), lambda i: (0, 0))],
        out_specs=pl.BlockSpec((_RB, fo), lambda i: (i, 0)),
        out_shape=jax.ShapeDtypeStruct((_NP, fo), jnp.float32),
    )(acc, acc, den, den, ea, bias, g, beta)


def _kemb_body(a_ref, b_ref, da_ref, db_ref, ea_ref, bias_ref, g_ref,
               beta_ref, h0_ref, sw_ref, sb_ref, o_ref):
    v = _norm_block(a_ref[0], b_ref[0], da_ref[0], db_ref[0], ea_ref[...],
                    bias_ref[...], g_ref[...], beta_ref[...])
    o_ref[...] = v + jnp.dot(h0_ref[...], sw_ref[...],
                             precision=_PREC) + sb_ref[...]


def _k_emb(acc, den, ea, bias, g, beta, h0, sw, sb, fh, fo):
    return pl.pallas_call(
        _kemb_body,
        grid=(_NP // _RB,),
        in_specs=[pl.BlockSpec((1, _RB, fh), lambda i: (0, i, 0)),
                  pl.BlockSpec((1, _RB, fh), lambda i: (1, i, 0)),
                  pl.BlockSpec((1, _RB, 16), lambda i: (0, i, 0)),
                  pl.BlockSpec((1, _RB, 16), lambda i: (1, i, 0)),
                  pl.BlockSpec((16, fh), lambda i: (0, 0)),
                  pl.BlockSpec((1, fo), lambda i: (0, 0)),
                  pl.BlockSpec((1, fo), lambda i: (0, 0)),
                  pl.BlockSpec((1, fo), lambda i: (0, 0)),
                  pl.BlockSpec((_RB, _HID), lambda i: (i, 0)),
                  pl.BlockSpec((_HID, fo), lambda i: (0, 0)),
                  pl.BlockSpec((1, fo), lambda i: (0, 0))],
        out_specs=pl.BlockSpec((_RB, fo), lambda i: (i, 0)),
        out_shape=jax.ShapeDtypeStruct((_NP, fo), jnp.float32),
    )(acc, acc, den, den, ea, bias, g, beta, h0, sw, sb)


def _kq_body(e_ref, fw1, fb1, fw2, fb2, mw1, mb1, mw2, mb2, bw,
             qf_ref, qm_ref):
    q = e_ref[...]
    f = jax.nn.relu(jnp.dot(q, fw1[...], precision=_PREC) + fb1[...])
    f = jax.nn.relu(jnp.dot(f, fw2[...], precision=_PREC) + fb2[...])
    qf_ref[...] = jnp.dot(f, bw[...], precision=_PREC)
    m = jax.nn.relu(jnp.dot(q, mw1[...], precision=_PREC) + mb1[...])
    m = jax.nn.relu(jnp.dot(m, mw2[...], precision=_PREC) + mb2[...])
    qm_ref[...] = jnp.dot(m, bw[...], precision=_PREC)


def _k_q(emb, fw1, fb1, fw2, fb2, mw1, mb1, mw2, mb2, bw, nq):
    return pl.pallas_call(
        _kq_body,
        in_specs=[pl.BlockSpec((nq, _EMB), lambda: (0, 0)),
                  pl.BlockSpec((_EMB, _HID), lambda: (0, 0)),
                  pl.BlockSpec((1, _HID), lambda: (0, 0)),
                  pl.BlockSpec((_HID, _HID), lambda: (0, 0)),
                  pl.BlockSpec((1, _HID), lambda: (0, 0)),
                  pl.BlockSpec((_EMB, _HID), lambda: (0, 0)),
                  pl.BlockSpec((1, _HID), lambda: (0, 0)),
                  pl.BlockSpec((_HID, _HID), lambda: (0, 0)),
                  pl.BlockSpec((1, _HID), lambda: (0, 0)),
                  pl.BlockSpec((_HID, _EMB), lambda: (0, 0))],
        out_specs=[pl.BlockSpec((nq, _EMB), lambda: (0, 0)),
                   pl.BlockSpec((nq, _EMB), lambda: (0, 0))],
        out_shape=(jax.ShapeDtypeStruct((nq, _EMB), jnp.float32),
                   jax.ShapeDtypeStruct((nq, _EMB), jnp.float32)),
    )(emb, fw1, fb1, fw2, fb2, mw1, mb1, mw2, mb2, bw)


def _kscore_body(qf_ref, qm_ref, e_ref, gen_ref, gd_ref, genq_ref, bb_ref,
                 fo_ref, mo_ref):
    emb = e_ref[...]
    dn = (((1,), (1,)), ((), ()))
    sf = lax.dot_general(qf_ref[...], emb, dn, precision=_PREC) + bb_ref[...]
    sm = lax.dot_general(qm_ref[...], emb, dn, precision=_PREC) + bb_ref[...]
    gd = gen_ref[0:1, :] - genq_ref[...]
    invalid = (gd < 0.5) | (gd > 2.0)
    male = gd_ref[0:1, :] == 1.0
    female = gd_ref[0:1, :] == 0.0
    fo_ref[...] = jnp.where(male & (~invalid), sf, _NEG_INF)
    mo_ref[...] = jnp.where(female & (~invalid), sm, _NEG_INF)


def _k_score(qf, qm, emb, genr8, gender8, genq, bb, nq):
    nb = _NP // _RB
    return pl.pallas_call(
        _kscore_body,
        grid=(nb,),
        in_specs=[pl.BlockSpec((nq, _EMB), lambda i: (0, 0)),
                  pl.BlockSpec((nq, _EMB), lambda i: (0, 0)),
                  pl.BlockSpec((_RB, _EMB), lambda i: (i, 0)),
                  pl.BlockSpec((8, _RB), lambda i: (0, i)),
                  pl.BlockSpec((8, _RB), lambda i: (0, i)),
                  pl.BlockSpec((nq, 1), lambda i: (0, 0)),
                  pl.BlockSpec((1, 1), lambda i: (0, 0))],
        out_specs=[pl.BlockSpec((nq, _RB), lambda i: (0, i)),
                   pl.BlockSpec((nq, _RB), lambda i: (0, i))],
        out_shape=(jax.ShapeDtypeStruct((nq, _NP), jnp.float32),
                   jax.ShapeDtypeStruct((nq, _NP), jnp.float32)),
    )(qf, qm, emb, genr8, gender8, genq, bb)


def _edge_chunks():
    npt = -(-_E // (16 * _K))
    npt = ((npt + 2) // 3) * 3
    return npt


def _sc_edge(tab, asb, adb, srcoff, dstoff, dst3, fh, heads):
    npt = _edge_chunks()
    nrt = _NP // 16
    nzc = nrt // _K + (1 if nrt % _K else 0)
    chw = min(_HID // heads, fh)
    ch = fh // chw
    mesh = plsc.VectorSubcoreMesh(core_axis_name="c", subcore_axis_name="s",
                                  num_cores=2, num_subcores=16)

    @functools.partial(
        pl.kernel,
        out_type=(jax.ShapeDtypeStruct((2, _NP, fh), jnp.float32),
                  jax.ShapeDtypeStruct((2, _NP, 16), jnp.float32)),
        mesh=mesh,
        compiler_params=pltpu.CompilerParams(use_tc_tiling_on_sc=False),
        scratch_types=[
            [pltpu.VMEM((_K,), jnp.int32)] * 3,
            [pltpu.VMEM((_K,), jnp.int32)] * 3,
            [pltpu.VMEM((_K,), jnp.int32)] * 3,
            [pltpu.VMEM((_K, 16), jnp.float32)] * 3,
            [pltpu.VMEM((_K, 16), jnp.float32)] * 3,
            [pltpu.VMEM((_K, 16), jnp.float32)] * 3,
            [pltpu.VMEM((_K, fh), jnp.float32)] * 3,
            pltpu.VMEM_SHARED((_NP, fh), jnp.float32),
            pltpu.VMEM_SHARED((_NP, 16), jnp.float32),
            [pltpu.SemaphoreType.DMA] * 3,
            [pltpu.SemaphoreType.DMA] * 3,
            [pltpu.SemaphoreType.DMA] * 3,
        ])
    def k(tab_h, as_h, ad_h, so_h, do_h, dr_h, acc_o, den_o, s2, d2, dr,
          av, bv, wv, rows, acc_sh, den_sh, si, sg, ss):
        c = lax.axis_index("c")
        s = lax.axis_index("s")
        zero16 = jnp.zeros((16,), jnp.float32)

        def zb(i, carry):
            for v in range(fh // 16):
                rows[0][i, pl.ds(16 * v, 16)] = zero16
            wv[0][i, :] = zero16
            return carry
        lax.fori_loop(0, _K, zb, 0)

        for q in range(nzc):
            base = s * nrt + min(q * _K, nrt - _K)
            pltpu.sync_copy(rows[0], acc_sh.at[pl.ds(base, _K)])
            pltpu.sync_copy(wv[0], den_sh.at[pl.ds(base, _K)])
        plsc.subcore_barrier()

        def i_copies(n, b):
            return (pltpu.make_async_copy(so_h.at[c, s, n], s2[b], si[b]),
                    pltpu.make_async_copy(do_h.at[c, s, n], d2[b], si[b]),
                    pltpu.make_async_copy(dr_h.at[s, n], dr[b], si[b]))

        def g_copies(n, b):
            return (pltpu.make_async_copy(as_h.at[s2[b]], av[b], sg[b]),
                    pltpu.make_async_copy(ad_h.at[d2[b]], bv[b], sg[b]),
                    pltpu.make_async_copy(tab_h.at[s2[b]], rows[b], sg[b]))

        def s_copies(n, b):
            return (pltpu.make_async_copy(wv[b], den_sh.at[dr[b]], ss[b]),
                    pltpu.make_async_copy(rows[b], acc_sh.at[dr[b]], ss[b]))

        def piece(n, b):
            bn = (b + 1) % 3
            bnn = (b + 2) % 3

            @pl.when(n >= 2)
            def _():
                for cp in s_copies(n - 2, bn):
                    cp.wait()

            @pl.when(n + 1 < npt)
            def _():
                for cp in i_copies(n + 1, bn):
                    cp.wait()
                for cp in g_copies(n + 1, bn):
                    cp.start()

            @pl.when(n + 2 < npt)
            def _():
                for cp in i_copies(n + 2, bnn):
                    cp.start()

            for cp in g_copies(n, b):
                cp.wait()

            def edge(k2, cy):
                t = av[b][k2, :] + bv[b][k2, :]
                t = jnp.maximum(t, 0.0) + 0.2 * jnp.minimum(t, 0.0)
                w = jnp.exp(t)
                wv[b][k2, :] = w
                for hh in range(ch):
                    wsc = w[hh]
                    for v2 in range(chw // 16):
                        sl = pl.ds(hh * chw + 16 * v2, 16)
                        rows[b][k2, sl] = rows[b][k2, sl] * wsc
                return cy
            lax.fori_loop(0, _K, edge, 0, unroll=2)

            pltpu.async_copy(wv[b], den_sh.at[dr[b]], ss[b], add=True)
            pltpu.async_copy(rows[b], acc_sh.at[dr[b]], ss[b], add=True)

        for cp in i_copies(0, 0):
            cp.start()
        for cp in i_copies(1, 1):
            cp.start()
        for cp in i_copies(0, 0):
            cp.wait()
        for cp in g_copies(0, 0):
            cp.start()

        def chunk3(t, carry):
            piece(3 * t, 0)
            piece(3 * t + 1, 1)
            piece(3 * t + 2, 2)
            return carry
        lax.fori_loop(0, npt // 3, chunk3, 0)

        for cp in s_copies(npt - 2, (npt - 2) % 3):
            cp.wait()
        for cp in s_copies(npt - 1, (npt - 1) % 3):
            cp.wait()
        plsc.subcore_barrier()
        rbase = s * nrt
        pltpu.sync_copy(acc_sh.at[pl.ds(rbase, nrt)],
                        acc_o.at[c, pl.ds(rbase, nrt)])
        pltpu.sync_copy(den_sh.at[pl.ds(rbase, nrt)],
                        den_o.at[c, pl.ds(rbase, nrt)])

    return k(tab, asb, adb, srcoff, dstoff, dst3)


def _expand_mats(heads, oph, fh):
    ea = np.zeros((16, fh), np.float32)
    chw = min(oph, fh)
    for h in range(fh // chw):
        ea[h, h * chw:(h + 1) * chw] = 1.0
    return jnp.asarray(ea)


def _pad16(v):
    return jnp.pad(v, ((0, 0), (0, 16 - v.shape[1])))


def kernel(x, edge_index, test_mask, gender, generation, W_in, b_in,
           gat_W_0, gat_as_0, gat_ad_0, gat_b_0, ln_g_0, ln_b_0,
           gat_W_1, gat_as_1, gat_ad_1, gat_b_1, ln_g_1, ln_b_1,
           gat_W_2, gat_as_2, gat_ad_2, gat_b_2, ln_g_2, ln_b_2,
           gat_W_3, gat_as_3, gat_ad_3, gat_b_3, ln_g_3, ln_b_3,
           skip_W, skip_b, fa_W1, fa_b1, fa_W2, fa_b2,
           mo_W1, mo_b1, mo_W2, mo_b2, bil_W, bil_b):
    f32 = jnp.float32
    nq = _NQ
    x_p = jnp.pad(x.astype(f32), ((0, _NP - _N), (0, 0)))
    src = edge_index[0].astype(jnp.int32)
    dst = edge_index[1].astype(jnp.int32)
    npt = _edge_chunks()
    ep = 16 * _K * npt
    srcp = jnp.concatenate([src, jnp.full((ep - _E,), _N, jnp.int32)])
    dstp = jnp.concatenate([dst, jnp.full((ep - _E,), _N, jnp.int32)])
    srcoff = jnp.stack([srcp, srcp + _NP]).reshape(2, 16, npt, _K)
    dstoff = jnp.stack([dstp, dstp + _NP]).reshape(2, 16, npt, _K)
    dst3 = dstp.reshape(16, npt, _K)

    h = _k_in(x_p, W_in.astype(f32), b_in.astype(f32).reshape(1, -1))
    h0 = h

    gat_W = [gat_W_0, gat_W_1, gat_W_2, gat_W_3]
    gat_as = [gat_as_0, gat_as_1, gat_as_2, gat_as_3]
    gat_ad = [gat_ad_0, gat_ad_1, gat_ad_2, gat_ad_3]
    gat_b = [gat_b_0, gat_b_1, gat_b_2, gat_b_3]
    ln_g = [ln_g_0, ln_g_1, ln_g_2, ln_g_3]
    ln_b = [ln_b_0, ln_b_1, ln_b_2, ln_b_3]

    emb = None
    for i in range(4):
        heads, oph = (8, _HID // 8) if i < 3 else (1, _EMB)
        fo = heads * oph
        fh = fo // 2
        w = gat_W[i].astype(f32)
        wt = jnp.stack([w[:, :fh], w[:, fh:]], 0)
        ws = jnp.einsum('fhd,hd->fh', w.reshape(_HID, heads, oph),
                        gat_as[i].astype(f32))
        wd = jnp.einsum('fhd,hd->fh', w.reshape(_HID, heads, oph),
                        gat_ad[i].astype(f32))
        nlh = heads // 2
        sbanks, dbanks = [], []
        for cc in range(2):
            hs = cc * nlh
            ncol = max(nlh, 1)
            sbanks.append(_pad16(ws[:, hs:hs + ncol]))
            dbanks.append(_pad16(wd[:, hs:hs + ncol]))
        wsb = jnp.stack(sbanks, 0)
        wdb = jnp.stack(dbanks, 0)
        tab, asb, adb = _k_dense(h, wt, wsb, wdb, fh)
        acc, den = _sc_edge(tab, asb, adb, srcoff, dstoff, dst3, fh, heads)
        ea = _expand_mats(heads, oph, fh)
        bias = gat_b[i].astype(f32).reshape(1, -1)
        g = ln_g[i].astype(f32).reshape(1, -1)
        beta = ln_b[i].astype(f32).reshape(1, -1)
        if i < 3:
            h = _k_norm(acc, den, ea, bias, g, beta, fh, fo)
        else:
            emb = _k_emb(acc, den, ea, bias, g, beta, h0,
                         skip_W.astype(f32), skip_b.astype(f32).reshape(1, -1),
                         fh, fo)

    qf, qm = _k_q(emb[:nq], fa_W1.astype(f32), fa_b1.astype(f32).reshape(1, -1),
                  fa_W2.astype(f32), fa_b2.astype(f32).reshape(1, -1),
                  mo_W1.astype(f32), mo_b1.astype(f32).reshape(1, -1),
                  mo_W2.astype(f32), mo_b2.astype(f32).reshape(1, -1),
                  bil_W.astype(f32), nq)

    gender_p = jnp.pad(gender.astype(f32), (0, _NP - _N), constant_values=5.0)
    genr_p = jnp.pad(generation.astype(f32), (0, _NP - _N))
    gender8 = jnp.broadcast_to(gender_p[None, :], (8, _NP))
    genr8 = jnp.broadcast_to(genr_p[None, :], (8, _NP))
    genq = generation.astype(f32)[:nq].reshape(nq, 1)
    bb = bil_b.astype(f32).reshape(1, 1)

    father, mother = _k_score(qf, qm, emb, genr8, gender8, genq, bb, nq)
    return father[:, :_N], mother[:, :_N], emb[:_N]


# speedup vs baseline: 1.1485x; 1.0478x over previous
import functools

import jax
import jax.numpy as jnp
import numpy as np
from jax import lax
from jax.experimental import pallas as pl
from jax.experimental.pallas import tpu as pltpu
from jax.experimental.pallas import tpu_sc as plsc

_N = 10000
_E = 320000
_NP = 10240
_RB = 2048
_K = 48
_HID = 256
_EMB = 128
_NEG_INF = -1e9
_NQ = 256
_PREC = lax.Precision.HIGHEST


def _kin_body(x_ref, w_ref, b_ref, o_ref):
    o_ref[...] = jax.nn.relu(
        jnp.dot(x_ref[...], w_ref[...], precision=_PREC) + b_ref[...])


def _k_in(x_p, w, b2):
    return pl.pallas_call(
        _kin_body,
        grid=(_NP // _RB,),
        in_specs=[pl.BlockSpec((_RB, _EMB), lambda i: (i, 0)),
                  pl.BlockSpec((_EMB, _HID), lambda i: (0, 0)),
                  pl.BlockSpec((1, _HID), lambda i: (0, 0))],
        out_specs=pl.BlockSpec((_RB, _HID), lambda i: (i, 0)),
        out_shape=jax.ShapeDtypeStruct((_NP, _HID), jnp.float32),
    )(x_p, w, b2)


def _kdense_body(h_ref, w_ref, wsd_ref, tab_ref, al_ref):
    h = h_ref[...]
    tab_ref[...] = jnp.dot(h, w_ref[0], precision=_PREC)
    al_ref[...] = jnp.dot(h, wsd_ref[0], precision=_PREC)


def _k_dense(h, wt, wsd, fh):
    nb = _NP // _RB
    return pl.pallas_call(
        _kdense_body,
        grid=(2, nb),
        in_specs=[pl.BlockSpec((_RB, _HID), lambda j, i: (i, 0)),
                  pl.BlockSpec((1, _HID, fh), lambda j, i: (j, 0, 0)),
                  pl.BlockSpec((1, _HID, 32), lambda j, i: (j, 0, 0))],
        out_specs=[pl.BlockSpec((_RB, fh), lambda j, i: (j * nb + i, 0)),
                   pl.BlockSpec((_RB, 32), lambda j, i: (j * nb + i, 0))],
        out_shape=(jax.ShapeDtypeStruct((2 * _NP, fh), jnp.float32),
                   jax.ShapeDtypeStruct((2 * _NP, 32), jnp.float32)),
    )(h, wt, wsd)


def _norm_block(a, b, da, db, ea, bias, g, beta):
    div_a = jnp.dot(1.0 / (da + 1e-16), ea, precision=_PREC)
    div_b = jnp.dot(1.0 / (db + 1e-16), ea, precision=_PREC)
    v = jnp.concatenate([a * div_a, b * div_b], axis=1) + bias
    mu = jnp.mean(v, axis=1, keepdims=True)
    var = jnp.mean((v - mu) ** 2, axis=1, keepdims=True)
    return (v - mu) / jnp.sqrt(var + 1e-5) * g + beta


def _knorm_body(a_ref, b_ref, da_ref, db_ref, ea_ref, bias_ref, g_ref,
                beta_ref, o_ref):
    v = _norm_block(a_ref[0], b_ref[0], da_ref[0], db_ref[0], ea_ref[...],
                    bias_ref[...], g_ref[...], beta_ref[...])
    o_ref[...] = jax.nn.relu(v)


def _k_norm(acc, den, ea, bias, g, beta, fh, fo):
    return pl.pallas_call(
        _knorm_body,
        grid=(_NP // _RB,),
        in_specs=[pl.BlockSpec((1, _RB, fh), lambda i: (0, i, 0)),
                  pl.BlockSpec((1, _RB, fh), lambda i: (1, i, 0)),
                  pl.BlockSpec((1, _RB, 16), lambda i: (0, i, 0)),
                  pl.BlockSpec((1, _RB, 16), lambda i: (1, i, 0)),
                  pl.BlockSpec((16, fh), lambda i: (0, 0)),
                  pl.BlockSpec((1, fo), lambda i: (0, 0)),
                  pl.BlockSpec((1, fo), lambda i: (0, 0)),
                  pl.BlockSpec((1, fo), lambda i: (0, 0))],
        out_specs=pl.BlockSpec((_RB, fo), lambda i: (i, 0)),
        out_shape=jax.ShapeDtypeStruct((_NP, fo), jnp.float32),
    )(acc, acc, den, den, ea, bias, g, beta)


def _kemb_body(a_ref, b_ref, da_ref, db_ref, ea_ref, bias_ref, g_ref,
               beta_ref, h0_ref, sw_ref, sb_ref, o_ref):
    v = _norm_block(a_ref[0], b_ref[0], da_ref[0], db_ref[0], ea_ref[...],
                    bias_ref[...], g_ref[...], beta_ref[...])
    o_ref[...] = v + jnp.dot(h0_ref[...], sw_ref[...],
                             precision=_PREC) + sb_ref[...]


def _k_emb(acc, den, ea, bias, g, beta, h0, sw, sb, fh, fo):
    return pl.pallas_call(
        _kemb_body,
        grid=(_NP // _RB,),
        in_specs=[pl.BlockSpec((1, _RB, fh), lambda i: (0, i, 0)),
                  pl.BlockSpec((1, _RB, fh), lambda i: (1, i, 0)),
                  pl.BlockSpec((1, _RB, 16), lambda i: (0, i, 0)),
                  pl.BlockSpec((1, _RB, 16), lambda i: (1, i, 0)),
                  pl.BlockSpec((16, fh), lambda i: (0, 0)),
                  pl.BlockSpec((1, fo), lambda i: (0, 0)),
                  pl.BlockSpec((1, fo), lambda i: (0, 0)),
                  pl.BlockSpec((1, fo), lambda i: (0, 0)),
                  pl.BlockSpec((_RB, _HID), lambda i: (i, 0)),
                  pl.BlockSpec((_HID, fo), lambda i: (0, 0)),
                  pl.BlockSpec((1, fo), lambda i: (0, 0))],
        out_specs=pl.BlockSpec((_RB, fo), lambda i: (i, 0)),
        out_shape=jax.ShapeDtypeStruct((_NP, fo), jnp.float32),
    )(acc, acc, den, den, ea, bias, g, beta, h0, sw, sb)


def _kq_body(e_ref, fw1, fb1, fw2, fb2, mw1, mb1, mw2, mb2, bw,
             qf_ref, qm_ref):
    q = e_ref[...]
    f = jax.nn.relu(jnp.dot(q, fw1[...], precision=_PREC) + fb1[...])
    f = jax.nn.relu(jnp.dot(f, fw2[...], precision=_PREC) + fb2[...])
    qf_ref[...] = jnp.dot(f, bw[...], precision=_PREC)
    m = jax.nn.relu(jnp.dot(q, mw1[...], precision=_PREC) + mb1[...])
    m = jax.nn.relu(jnp.dot(m, mw2[...], precision=_PREC) + mb2[...])
    qm_ref[...] = jnp.dot(m, bw[...], precision=_PREC)


def _k_q(emb, fw1, fb1, fw2, fb2, mw1, mb1, mw2, mb2, bw, nq):
    return pl.pallas_call(
        _kq_body,
        in_specs=[pl.BlockSpec((nq, _EMB), lambda: (0, 0)),
                  pl.BlockSpec((_EMB, _HID), lambda: (0, 0)),
                  pl.BlockSpec((1, _HID), lambda: (0, 0)),
                  pl.BlockSpec((_HID, _HID), lambda: (0, 0)),
                  pl.BlockSpec((1, _HID), lambda: (0, 0)),
                  pl.BlockSpec((_EMB, _HID), lambda: (0, 0)),
                  pl.BlockSpec((1, _HID), lambda: (0, 0)),
                  pl.BlockSpec((_HID, _HID), lambda: (0, 0)),
                  pl.BlockSpec((1, _HID), lambda: (0, 0)),
                  pl.BlockSpec((_HID, _EMB), lambda: (0, 0))],
        out_specs=[pl.BlockSpec((nq, _EMB), lambda: (0, 0)),
                   pl.BlockSpec((nq, _EMB), lambda: (0, 0))],
        out_shape=(jax.ShapeDtypeStruct((nq, _EMB), jnp.float32),
                   jax.ShapeDtypeStruct((nq, _EMB), jnp.float32)),
    )(emb, fw1, fb1, fw2, fb2, mw1, mb1, mw2, mb2, bw)


def _kscore_body(qf_ref, qm_ref, e_ref, gen_ref, gd_ref, genq_ref, bb_ref,
                 fo_ref, mo_ref):
    emb = e_ref[...]
    dn = (((1,), (1,)), ((), ()))
    sf = lax.dot_general(qf_ref[...], emb, dn, precision=_PREC) + bb_ref[...]
    sm = lax.dot_general(qm_ref[...], emb, dn, precision=_PREC) + bb_ref[...]
    gd = gen_ref[0:1, :] - genq_ref[...]
    invalid = (gd < 0.5) | (gd > 2.0)
    male = gd_ref[0:1, :] == 1.0
    female = gd_ref[0:1, :] == 0.0
    fo_ref[...] = jnp.where(male & (~invalid), sf, _NEG_INF)
    mo_ref[...] = jnp.where(female & (~invalid), sm, _NEG_INF)


def _k_score(qf, qm, emb, genr8, gender8, genq, bb, nq):
    nb = _NP // _RB
    return pl.pallas_call(
        _kscore_body,
        grid=(nb,),
        in_specs=[pl.BlockSpec((nq, _EMB), lambda i: (0, 0)),
                  pl.BlockSpec((nq, _EMB), lambda i: (0, 0)),
                  pl.BlockSpec((_RB, _EMB), lambda i: (i, 0)),
                  pl.BlockSpec((8, _RB), lambda i: (0, i)),
                  pl.BlockSpec((8, _RB), lambda i: (0, i)),
                  pl.BlockSpec((nq, 1), lambda i: (0, 0)),
                  pl.BlockSpec((1, 1), lambda i: (0, 0))],
        out_specs=[pl.BlockSpec((nq, _RB), lambda i: (0, i)),
                   pl.BlockSpec((nq, _RB), lambda i: (0, i))],
        out_shape=(jax.ShapeDtypeStruct((nq, _NP), jnp.float32),
                   jax.ShapeDtypeStruct((nq, _NP), jnp.float32)),
    )(qf, qm, emb, genr8, gender8, genq, bb)


def _edge_chunks():
    npt = -(-_E // (16 * _K))
    npt = ((npt + 2) // 3) * 3
    return npt


def _sc_edge(tab, al, srcoff, dstoff, dst3, fh, heads):
    npt = _edge_chunks()
    nrt = _NP // 16
    nzc = nrt // _K + (1 if nrt % _K else 0)
    chw = min(_HID // heads, fh)
    ch = fh // chw
    mesh = plsc.VectorSubcoreMesh(core_axis_name="c", subcore_axis_name="s",
                                  num_cores=2, num_subcores=16)

    @functools.partial(
        pl.kernel,
        out_type=(jax.ShapeDtypeStruct((2, _NP, fh), jnp.float32),
                  jax.ShapeDtypeStruct((2, _NP, 16), jnp.float32)),
        mesh=mesh,
        compiler_params=pltpu.CompilerParams(use_tc_tiling_on_sc=False),
        scratch_types=[
            [pltpu.VMEM((_K,), jnp.int32)] * 3,
            [pltpu.VMEM((_K,), jnp.int32)] * 3,
            [pltpu.VMEM((_K,), jnp.int32)] * 3,
            [pltpu.VMEM((_K, 32), jnp.float32)] * 3,
            [pltpu.VMEM((_K, 32), jnp.float32)] * 3,
            [pltpu.VMEM((_K, 16), jnp.float32)] * 3,
            [pltpu.VMEM((_K, fh), jnp.float32)] * 3,
            pltpu.VMEM_SHARED((_NP, fh), jnp.float32),
            pltpu.VMEM_SHARED((_NP, 16), jnp.float32),
            [pltpu.SemaphoreType.DMA] * 3,
            [pltpu.SemaphoreType.DMA] * 3,
            [pltpu.SemaphoreType.DMA] * 3,
        ])
    def k(tab_h, al_h, so_h, do_h, dr_h, acc_o, den_o, s2, d2, dr,
          av, bv, wv, rows, acc_sh, den_sh, si, sg, ss):
        c = lax.axis_index("c")
        s = lax.axis_index("s")
        zero16 = jnp.zeros((16,), jnp.float32)

        def zb(i, carry):
            for v in range(fh // 16):
                rows[0][i, pl.ds(16 * v, 16)] = zero16
            wv[0][i, :] = zero16
            return carry
        lax.fori_loop(0, _K, zb, 0)

        for q in range(nzc):
            base = s * nrt + min(q * _K, nrt - _K)
            pltpu.sync_copy(rows[0], acc_sh.at[pl.ds(base, _K)])
            pltpu.sync_copy(wv[0], den_sh.at[pl.ds(base, _K)])
        plsc.subcore_barrier()

        def i_copies(n, b):
            return (pltpu.make_async_copy(so_h.at[c, s, n], s2[b], si[b]),
                    pltpu.make_async_copy(do_h.at[c, s, n], d2[b], si[b]),
                    pltpu.make_async_copy(dr_h.at[s, n], dr[b], si[b]))

        def g_copies(n, b):
            return (pltpu.make_async_copy(al_h.at[s2[b]], av[b], sg[b]),
                    pltpu.make_async_copy(al_h.at[d2[b]], bv[b], sg[b]),
                    pltpu.make_async_copy(tab_h.at[s2[b]], rows[b], sg[b]))

        def s_copies(n, b):
            return (pltpu.make_async_copy(wv[b], den_sh.at[dr[b]], ss[b]),
                    pltpu.make_async_copy(rows[b], acc_sh.at[dr[b]], ss[b]))

        def piece(n, b):
            bn = (b + 1) % 3
            bnn = (b + 2) % 3

            @pl.when(n >= 2)
            def _():
                for cp in s_copies(n - 2, bn):
                    cp.wait()

            @pl.when(n + 1 < npt)
            def _():
                for cp in i_copies(n + 1, bn):
                    cp.wait()
                for cp in g_copies(n + 1, bn):
                    cp.start()

            @pl.when(n + 2 < npt)
            def _():
                for cp in i_copies(n + 2, bnn):
                    cp.start()

            for cp in g_copies(n, b):
                cp.wait()

            def wcomp(k2, cy):
                t = av[b][k2, 0:16] + bv[b][k2, 16:32]
                t = jnp.maximum(t, 0.2 * t)
                wv[b][k2, :] = jnp.exp(t)
                return cy
            lax.fori_loop(0, _K, wcomp, 0, unroll=8)

            def mul(k2, cy):
                w = wv[b][k2, :]
                for hh in range(ch):
                    wsc = w[hh]
                    for v2 in range(chw // 16):
                        sl = pl.ds(hh * chw + 16 * v2, 16)
                        rows[b][k2, sl] = rows[b][k2, sl] * wsc
                return cy
            lax.fori_loop(0, _K, mul, 0, unroll=2)

            pltpu.async_copy(wv[b], den_sh.at[dr[b]], ss[b], add=True)
            pltpu.async_copy(rows[b], acc_sh.at[dr[b]], ss[b], add=True)

        for cp in i_copies(0, 0):
            cp.start()
        for cp in i_copies(1, 1):
            cp.start()
        for cp in i_copies(0, 0):
            cp.wait()
        for cp in g_copies(0, 0):
            cp.start()

        def chunk3(t, carry):
            piece(3 * t, 0)
            piece(3 * t + 1, 1)
            piece(3 * t + 2, 2)
            return carry
        lax.fori_loop(0, npt // 3, chunk3, 0)

        for cp in s_copies(npt - 2, (npt - 2) % 3):
            cp.wait()
        for cp in s_copies(npt - 1, (npt - 1) % 3):
            cp.wait()
        plsc.subcore_barrier()
        rbase = s * nrt
        pltpu.sync_copy(acc_sh.at[pl.ds(rbase, nrt)],
                        acc_o.at[c, pl.ds(rbase, nrt)])
        pltpu.sync_copy(den_sh.at[pl.ds(rbase, nrt)],
                        den_o.at[c, pl.ds(rbase, nrt)])

    return k(tab, al, srcoff, dstoff, dst3)


def _expand_mats(heads, oph, fh):
    ea = np.zeros((16, fh), np.float32)
    chw = min(oph, fh)
    for h in range(fh // chw):
        ea[h, h * chw:(h + 1) * chw] = 1.0
    return jnp.asarray(ea)


def _pad16(v):
    return jnp.pad(v, ((0, 0), (0, 16 - v.shape[1])))


def kernel(x, edge_index, test_mask, gender, generation, W_in, b_in,
           gat_W_0, gat_as_0, gat_ad_0, gat_b_0, ln_g_0, ln_b_0,
           gat_W_1, gat_as_1, gat_ad_1, gat_b_1, ln_g_1, ln_b_1,
           gat_W_2, gat_as_2, gat_ad_2, gat_b_2, ln_g_2, ln_b_2,
           gat_W_3, gat_as_3, gat_ad_3, gat_b_3, ln_g_3, ln_b_3,
           skip_W, skip_b, fa_W1, fa_b1, fa_W2, fa_b2,
           mo_W1, mo_b1, mo_W2, mo_b2, bil_W, bil_b):
    f32 = jnp.float32
    nq = _NQ
    x_p = jnp.pad(x.astype(f32), ((0, _NP - _N), (0, 0)))
    src = edge_index[0].astype(jnp.int32)
    dst = edge_index[1].astype(jnp.int32)
    npt = _edge_chunks()
    ep = 16 * _K * npt
    srcp = jnp.concatenate([src, jnp.full((ep - _E,), _N, jnp.int32)])
    dstp = jnp.concatenate([dst, jnp.full((ep - _E,), _N, jnp.int32)])
    srcoff = jnp.stack([srcp, srcp + _NP]).reshape(2, 16, npt, _K)
    dstoff = jnp.stack([dstp, dstp + _NP]).reshape(2, 16, npt, _K)
    dst3 = dstp.reshape(16, npt, _K)

    h = _k_in(x_p, W_in.astype(f32), b_in.astype(f32).reshape(1, -1))
    h0 = h

    gat_W = [gat_W_0, gat_W_1, gat_W_2, gat_W_3]
    gat_as = [gat_as_0, gat_as_1, gat_as_2, gat_as_3]
    gat_ad = [gat_ad_0, gat_ad_1, gat_ad_2, gat_ad_3]
    gat_b = [gat_b_0, gat_b_1, gat_b_2, gat_b_3]
    ln_g = [ln_g_0, ln_g_1, ln_g_2, ln_g_3]
    ln_b = [ln_b_0, ln_b_1, ln_b_2, ln_b_3]

    emb = None
    for i in range(4):
        heads, oph = (8, _HID // 8) if i < 3 else (1, _EMB)
        fo = heads * oph
        fh = fo // 2
        w = gat_W[i].astype(f32)
        wt = jnp.stack([w[:, :fh], w[:, fh:]], 0)
        ws = jnp.einsum('fhd,hd->fh', w.reshape(_HID, heads, oph),
                        gat_as[i].astype(f32))
        wd = jnp.einsum('fhd,hd->fh', w.reshape(_HID, heads, oph),
                        gat_ad[i].astype(f32))
        nlh = heads // 2
        banks = []
        for cc in range(2):
            hs = cc * nlh
            ncol = max(nlh, 1)
            banks.append(jnp.concatenate(
                [_pad16(ws[:, hs:hs + ncol]), _pad16(wd[:, hs:hs + ncol])],
                axis=1))
        wsd = jnp.stack(banks, 0)
        tab, al = _k_dense(h, wt, wsd, fh)
        acc, den = _sc_edge(tab, al, srcoff, dstoff, dst3, fh, heads)
        ea = _expand_mats(heads, oph, fh)
        bias = gat_b[i].astype(f32).reshape(1, -1)
        g = ln_g[i].astype(f32).reshape(1, -1)
        beta = ln_b[i].astype(f32).reshape(1, -1)
        if i < 3:
            h = _k_norm(acc, den, ea, bias, g, beta, fh, fo)
        else:
            emb = _k_emb(acc, den, ea, bias, g, beta, h0,
                         skip_W.astype(f32), skip_b.astype(f32).reshape(1, -1),
                         fh, fo)

    qf, qm = _k_q(emb[:nq], fa_W1.astype(f32), fa_b1.astype(f32).reshape(1, -1),
                  fa_W2.astype(f32), fa_b2.astype(f32).reshape(1, -1),
                  mo_W1.astype(f32), mo_b1.astype(f32).reshape(1, -1),
                  mo_W2.astype(f32), mo_b2.astype(f32).reshape(1, -1),
                  bil_W.astype(f32), nq)

    gender_p = jnp.pad(gender.astype(f32), (0, _NP - _N), constant_values=5.0)
    genr_p = jnp.pad(generation.astype(f32), (0, _NP - _N))
    gender8 = jnp.broadcast_to(gender_p[None, :], (8, _NP))
    genr8 = jnp.broadcast_to(genr_p[None, :], (8, _NP))
    genq = generation.astype(f32)[:nq].reshape(nq, 1)
    bb = bil_b.astype(f32).reshape(1, 1)

    father, mother = _k_score(qf, qm, emb, genr8, gender8, genq, bb, nq)
    return father[:, :_N], mother[:, :_N], emb[:_N]


# speedup vs baseline: 1.6551x; 1.4410x over previous
import functools

import jax
import jax.numpy as jnp
import numpy as np
from jax import lax
from jax.experimental import pallas as pl
from jax.experimental.pallas import tpu as pltpu
from jax.experimental.pallas import tpu_sc as plsc

_N = 10000
_E = 320000
_NP = 10240
_RB = 2048
_K = 48
_HID = 256
_EMB = 128
_NEG_INF = -1e9
_NQ = 256
_PREC = lax.Precision.HIGHEST


def _kin_body(x_ref, w_ref, b_ref, o_ref):
    o_ref[...] = jax.nn.relu(
        jnp.dot(x_ref[...], w_ref[...], precision=_PREC) + b_ref[...])


def _k_in(x_p, w, b2):
    return pl.pallas_call(
        _kin_body,
        grid=(_NP // _RB,),
        in_specs=[pl.BlockSpec((_RB, _EMB), lambda i: (i, 0)),
                  pl.BlockSpec((_EMB, _HID), lambda i: (0, 0)),
                  pl.BlockSpec((1, _HID), lambda i: (0, 0))],
        out_specs=pl.BlockSpec((_RB, _HID), lambda i: (i, 0)),
        out_shape=jax.ShapeDtypeStruct((_NP, _HID), jnp.float32),
    )(x_p, w, b2)


def _kdense_body(h_ref, w_ref, wsd_ref, tab_ref, al_ref):
    h = h_ref[...]
    tab_ref[...] = jnp.dot(h, w_ref[0], precision=_PREC)
    al_ref[...] = jnp.dot(h, wsd_ref[0], precision=_PREC)


def _k_dense(h, wt, wsd, fh):
    nb = _NP // _RB
    return pl.pallas_call(
        _kdense_body,
        grid=(2, nb),
        in_specs=[pl.BlockSpec((_RB, _HID), lambda j, i: (i, 0)),
                  pl.BlockSpec((1, _HID, fh), lambda j, i: (j, 0, 0)),
                  pl.BlockSpec((1, _HID, 32), lambda j, i: (j, 0, 0))],
        out_specs=[pl.BlockSpec((_RB, fh), lambda j, i: (j * nb + i, 0)),
                   pl.BlockSpec((_RB, 32), lambda j, i: (j * nb + i, 0))],
        out_shape=(jax.ShapeDtypeStruct((2 * _NP, fh), jnp.float32),
                   jax.ShapeDtypeStruct((2 * _NP, 32), jnp.float32)),
    )(h, wt, wsd)


def _norm_block(a, b, da, db, ea, bias, g, beta):
    div_a = jnp.dot(1.0 / (da + 1e-16), ea, precision=_PREC)
    div_b = jnp.dot(1.0 / (db + 1e-16), ea, precision=_PREC)
    v = jnp.concatenate([a * div_a, b * div_b], axis=1) + bias
    mu = jnp.mean(v, axis=1, keepdims=True)
    var = jnp.mean((v - mu) ** 2, axis=1, keepdims=True)
    return (v - mu) / jnp.sqrt(var + 1e-5) * g + beta


def _knorm_body(a_ref, b_ref, da_ref, db_ref, ea_ref, bias_ref, g_ref,
                beta_ref, o_ref):
    v = _norm_block(a_ref[0], b_ref[0], da_ref[0], db_ref[0], ea_ref[...],
                    bias_ref[...], g_ref[...], beta_ref[...])
    o_ref[...] = jax.nn.relu(v)


def _k_norm(acc, den, ea, bias, g, beta, fh, fo):
    return pl.pallas_call(
        _knorm_body,
        grid=(_NP // _RB,),
        in_specs=[pl.BlockSpec((1, _RB, fh), lambda i: (0, i, 0)),
                  pl.BlockSpec((1, _RB, fh), lambda i: (1, i, 0)),
                  pl.BlockSpec((1, _RB, 16), lambda i: (0, i, 0)),
                  pl.BlockSpec((1, _RB, 16), lambda i: (1, i, 0)),
                  pl.BlockSpec((16, fh), lambda i: (0, 0)),
                  pl.BlockSpec((1, fo), lambda i: (0, 0)),
                  pl.BlockSpec((1, fo), lambda i: (0, 0)),
                  pl.BlockSpec((1, fo), lambda i: (0, 0))],
        out_specs=pl.BlockSpec((_RB, fo), lambda i: (i, 0)),
        out_shape=jax.ShapeDtypeStruct((_NP, fo), jnp.float32),
    )(acc, acc, den, den, ea, bias, g, beta)


def _kemb_body(a_ref, b_ref, da_ref, db_ref, ea_ref, bias_ref, g_ref,
               beta_ref, h0_ref, sw_ref, sb_ref, o_ref):
    v = _norm_block(a_ref[0], b_ref[0], da_ref[0], db_ref[0], ea_ref[...],
                    bias_ref[...], g_ref[...], beta_ref[...])
    o_ref[...] = v + jnp.dot(h0_ref[...], sw_ref[...],
                             precision=_PREC) + sb_ref[...]


def _k_emb(acc, den, ea, bias, g, beta, h0, sw, sb, fh, fo):
    return pl.pallas_call(
        _kemb_body,
        grid=(_NP // _RB,),
        in_specs=[pl.BlockSpec((1, _RB, fh), lambda i: (0, i, 0)),
                  pl.BlockSpec((1, _RB, fh), lambda i: (1, i, 0)),
                  pl.BlockSpec((1, _RB, 16), lambda i: (0, i, 0)),
                  pl.BlockSpec((1, _RB, 16), lambda i: (1, i, 0)),
                  pl.BlockSpec((16, fh), lambda i: (0, 0)),
                  pl.BlockSpec((1, fo), lambda i: (0, 0)),
                  pl.BlockSpec((1, fo), lambda i: (0, 0)),
                  pl.BlockSpec((1, fo), lambda i: (0, 0)),
                  pl.BlockSpec((_RB, _HID), lambda i: (i, 0)),
                  pl.BlockSpec((_HID, fo), lambda i: (0, 0)),
                  pl.BlockSpec((1, fo), lambda i: (0, 0))],
        out_specs=pl.BlockSpec((_RB, fo), lambda i: (i, 0)),
        out_shape=jax.ShapeDtypeStruct((_NP, fo), jnp.float32),
    )(acc, acc, den, den, ea, bias, g, beta, h0, sw, sb)


def _kq_body(e_ref, fw1, fb1, fw2, fb2, mw1, mb1, mw2, mb2, bw,
             qf_ref, qm_ref):
    q = e_ref[...]
    f = jax.nn.relu(jnp.dot(q, fw1[...], precision=_PREC) + fb1[...])
    f = jax.nn.relu(jnp.dot(f, fw2[...], precision=_PREC) + fb2[...])
    qf_ref[...] = jnp.dot(f, bw[...], precision=_PREC)
    m = jax.nn.relu(jnp.dot(q, mw1[...], precision=_PREC) + mb1[...])
    m = jax.nn.relu(jnp.dot(m, mw2[...], precision=_PREC) + mb2[...])
    qm_ref[...] = jnp.dot(m, bw[...], precision=_PREC)


def _k_q(emb, fw1, fb1, fw2, fb2, mw1, mb1, mw2, mb2, bw, nq):
    return pl.pallas_call(
        _kq_body,
        in_specs=[pl.BlockSpec((nq, _EMB), lambda: (0, 0)),
                  pl.BlockSpec((_EMB, _HID), lambda: (0, 0)),
                  pl.BlockSpec((1, _HID), lambda: (0, 0)),
                  pl.BlockSpec((_HID, _HID), lambda: (0, 0)),
                  pl.BlockSpec((1, _HID), lambda: (0, 0)),
                  pl.BlockSpec((_EMB, _HID), lambda: (0, 0)),
                  pl.BlockSpec((1, _HID), lambda: (0, 0)),
                  pl.BlockSpec((_HID, _HID), lambda: (0, 0)),
                  pl.BlockSpec((1, _HID), lambda: (0, 0)),
                  pl.BlockSpec((_HID, _EMB), lambda: (0, 0))],
        out_specs=[pl.BlockSpec((nq, _EMB), lambda: (0, 0)),
                   pl.BlockSpec((nq, _EMB), lambda: (0, 0))],
        out_shape=(jax.ShapeDtypeStruct((nq, _EMB), jnp.float32),
                   jax.ShapeDtypeStruct((nq, _EMB), jnp.float32)),
    )(emb, fw1, fb1, fw2, fb2, mw1, mb1, mw2, mb2, bw)


def _kscore_body(qf_ref, qm_ref, e_ref, gen_ref, gd_ref, genq_ref, bb_ref,
                 fo_ref, mo_ref):
    emb = e_ref[...]
    dn = (((1,), (1,)), ((), ()))
    sf = lax.dot_general(qf_ref[...], emb, dn, precision=_PREC) + bb_ref[...]
    sm = lax.dot_general(qm_ref[...], emb, dn, precision=_PREC) + bb_ref[...]
    gd = gen_ref[0:1, :] - genq_ref[...]
    invalid = (gd < 0.5) | (gd > 2.0)
    male = gd_ref[0:1, :] == 1.0
    female = gd_ref[0:1, :] == 0.0
    fo_ref[...] = jnp.where(male & (~invalid), sf, _NEG_INF)
    mo_ref[...] = jnp.where(female & (~invalid), sm, _NEG_INF)


def _k_score(qf, qm, emb, genr8, gender8, genq, bb, nq):
    nb = _NP // _RB
    return pl.pallas_call(
        _kscore_body,
        grid=(nb,),
        in_specs=[pl.BlockSpec((nq, _EMB), lambda i: (0, 0)),
                  pl.BlockSpec((nq, _EMB), lambda i: (0, 0)),
                  pl.BlockSpec((_RB, _EMB), lambda i: (i, 0)),
                  pl.BlockSpec((8, _RB), lambda i: (0, i)),
                  pl.BlockSpec((8, _RB), lambda i: (0, i)),
                  pl.BlockSpec((nq, 1), lambda i: (0, 0)),
                  pl.BlockSpec((1, 1), lambda i: (0, 0))],
        out_specs=[pl.BlockSpec((nq, _RB), lambda i: (0, i)),
                   pl.BlockSpec((nq, _RB), lambda i: (0, i))],
        out_shape=(jax.ShapeDtypeStruct((nq, _NP), jnp.float32),
                   jax.ShapeDtypeStruct((nq, _NP), jnp.float32)),
    )(qf, qm, emb, genr8, gender8, genq, bb)


def _edge_chunks():
    npt = -(-_E // (16 * _K))
    npt = ((npt + 2) // 3) * 3
    return npt


def _sc_edge(tab, al, srcoff, dstoff, dst3, fh, heads):
    npt = _edge_chunks()
    nrt = _NP // 16
    nzc = nrt // _K + (1 if nrt % _K else 0)
    chw = min(_HID // heads, fh)
    ch = fh // chw
    mesh = plsc.VectorSubcoreMesh(core_axis_name="c", subcore_axis_name="s",
                                  num_cores=2, num_subcores=16)

    @functools.partial(
        pl.kernel,
        out_type=(jax.ShapeDtypeStruct((2, _NP, fh), jnp.float32),
                  jax.ShapeDtypeStruct((2, _NP, 16), jnp.float32)),
        mesh=mesh,
        compiler_params=pltpu.CompilerParams(use_tc_tiling_on_sc=False),
        scratch_types=[
            [pltpu.VMEM((_K,), jnp.int32)] * 3,
            [pltpu.VMEM((_K,), jnp.int32)] * 3,
            [pltpu.VMEM((_K,), jnp.int32)] * 3,
            [pltpu.VMEM((_K, 32), jnp.float32)] * 3,
            [pltpu.VMEM((_K, 32), jnp.float32)] * 3,
            [pltpu.VMEM((_K, 16), jnp.float32)] * 3,
            [pltpu.VMEM((_K, fh), jnp.float32)] * 3,
            pltpu.VMEM_SHARED((_NP, fh), jnp.float32),
            pltpu.VMEM_SHARED((_NP, 16), jnp.float32),
            [pltpu.SemaphoreType.DMA] * 3,
            [pltpu.SemaphoreType.DMA] * 3,
            [pltpu.SemaphoreType.DMA] * 3,
        ])
    def k(tab_h, al_h, so_h, do_h, dr_h, acc_o, den_o, s2, d2, dr,
          av, bv, wv, rows, acc_sh, den_sh, si, sg, ss):
        c = lax.axis_index("c")
        s = lax.axis_index("s")
        zero16 = jnp.zeros((16,), jnp.float32)

        def zb(i, carry):
            for v in range(fh // 16):
                rows[0][i, pl.ds(16 * v, 16)] = zero16
            wv[0][i, :] = zero16
            return carry
        lax.fori_loop(0, _K, zb, 0)

        for q in range(nzc):
            base = s * nrt + min(q * _K, nrt - _K)
            pltpu.sync_copy(rows[0], acc_sh.at[pl.ds(base, _K)])
            pltpu.sync_copy(wv[0], den_sh.at[pl.ds(base, _K)])
        plsc.subcore_barrier()

        def i_copies(n, b):
            return (pltpu.make_async_copy(so_h.at[c, s, n], s2[b], si[b]),
                    pltpu.make_async_copy(do_h.at[c, s, n], d2[b], si[b]),
                    pltpu.make_async_copy(dr_h.at[s, n], dr[b], si[b]))

        def g_copies(n, b):
            return (pltpu.make_async_copy(al_h.at[s2[b]], av[b], sg[b]),
                    pltpu.make_async_copy(al_h.at[d2[b]], bv[b], sg[b]),
                    pltpu.make_async_copy(tab_h.at[s2[b]], rows[b], sg[b]))

        def s_copies(n, b):
            return (pltpu.make_async_copy(wv[b], den_sh.at[dr[b]], ss[b]),
                    pltpu.make_async_copy(rows[b], acc_sh.at[dr[b]], ss[b]))

        def piece(n, b):
            bn = (b + 1) % 3
            bnn = (b + 2) % 3

            @pl.when(n >= 2)
            def _():
                for cp in s_copies(n - 2, bn):
                    cp.wait()

            @pl.when(n + 1 < npt)
            def _():
                for cp in i_copies(n + 1, bn):
                    cp.wait()
                for cp in g_copies(n + 1, bn):
                    cp.start()

            @pl.when(n + 2 < npt)
            def _():
                for cp in i_copies(n + 2, bnn):
                    cp.start()

            for cp in g_copies(n, b):
                cp.wait()

            @plsc.parallel_loop(0, _K, unroll=8)
            def _(k2):
                t = av[b][k2, 0:16] + bv[b][k2, 16:32]
                t = jnp.maximum(t, 0.2 * t)
                wv[b][k2, :] = jnp.exp(t)

            @plsc.parallel_loop(0, _K, unroll=2)
            def _(k2):
                w = wv[b][k2, :]
                for hh in range(ch):
                    wsc = w[hh]
                    for v2 in range(chw // 16):
                        sl = pl.ds(hh * chw + 16 * v2, 16)
                        rows[b][k2, sl] = rows[b][k2, sl] * wsc

            pltpu.async_copy(wv[b], den_sh.at[dr[b]], ss[b], add=True)
            pltpu.async_copy(rows[b], acc_sh.at[dr[b]], ss[b], add=True)

        for cp in i_copies(0, 0):
            cp.start()
        for cp in i_copies(1, 1):
            cp.start()
        for cp in i_copies(0, 0):
            cp.wait()
        for cp in g_copies(0, 0):
            cp.start()

        def chunk3(t, carry):
            piece(3 * t, 0)
            piece(3 * t + 1, 1)
            piece(3 * t + 2, 2)
            return carry
        lax.fori_loop(0, npt // 3, chunk3, 0)

        for cp in s_copies(npt - 2, (npt - 2) % 3):
            cp.wait()
        for cp in s_copies(npt - 1, (npt - 1) % 3):
            cp.wait()
        plsc.subcore_barrier()
        rbase = s * nrt
        pltpu.sync_copy(acc_sh.at[pl.ds(rbase, nrt)],
                        acc_o.at[c, pl.ds(rbase, nrt)])
        pltpu.sync_copy(den_sh.at[pl.ds(rbase, nrt)],
                        den_o.at[c, pl.ds(rbase, nrt)])

    return k(tab, al, srcoff, dstoff, dst3)


def _expand_mats(heads, oph, fh):
    ea = np.zeros((16, fh), np.float32)
    chw = min(oph, fh)
    for h in range(fh // chw):
        ea[h, h * chw:(h + 1) * chw] = 1.0
    return jnp.asarray(ea)


def _pad16(v):
    return jnp.pad(v, ((0, 0), (0, 16 - v.shape[1])))


def kernel(x, edge_index, test_mask, gender, generation, W_in, b_in,
           gat_W_0, gat_as_0, gat_ad_0, gat_b_0, ln_g_0, ln_b_0,
           gat_W_1, gat_as_1, gat_ad_1, gat_b_1, ln_g_1, ln_b_1,
           gat_W_2, gat_as_2, gat_ad_2, gat_b_2, ln_g_2, ln_b_2,
           gat_W_3, gat_as_3, gat_ad_3, gat_b_3, ln_g_3, ln_b_3,
           skip_W, skip_b, fa_W1, fa_b1, fa_W2, fa_b2,
           mo_W1, mo_b1, mo_W2, mo_b2, bil_W, bil_b):
    f32 = jnp.float32
    nq = _NQ
    x_p = jnp.pad(x.astype(f32), ((0, _NP - _N), (0, 0)))
    src = edge_index[0].astype(jnp.int32)
    dst = edge_index[1].astype(jnp.int32)
    npt = _edge_chunks()
    ep = 16 * _K * npt
    srcp = jnp.concatenate([src, jnp.full((ep - _E,), _N, jnp.int32)])
    dstp = jnp.concatenate([dst, jnp.full((ep - _E,), _N, jnp.int32)])
    srcoff = jnp.stack([srcp, srcp + _NP]).reshape(2, 16, npt, _K)
    dstoff = jnp.stack([dstp, dstp + _NP]).reshape(2, 16, npt, _K)
    dst3 = dstp.reshape(16, npt, _K)

    h = _k_in(x_p, W_in.astype(f32), b_in.astype(f32).reshape(1, -1))
    h0 = h

    gat_W = [gat_W_0, gat_W_1, gat_W_2, gat_W_3]
    gat_as = [gat_as_0, gat_as_1, gat_as_2, gat_as_3]
    gat_ad = [gat_ad_0, gat_ad_1, gat_ad_2, gat_ad_3]
    gat_b = [gat_b_0, gat_b_1, gat_b_2, gat_b_3]
    ln_g = [ln_g_0, ln_g_1, ln_g_2, ln_g_3]
    ln_b = [ln_b_0, ln_b_1, ln_b_2, ln_b_3]

    emb = None
    for i in range(4):
        heads, oph = (8, _HID // 8) if i < 3 else (1, _EMB)
        fo = heads * oph
        fh = fo // 2
        w = gat_W[i].astype(f32)
        wt = jnp.stack([w[:, :fh], w[:, fh:]], 0)
        ws = jnp.einsum('fhd,hd->fh', w.reshape(_HID, heads, oph),
                        gat_as[i].astype(f32))
        wd = jnp.einsum('fhd,hd->fh', w.reshape(_HID, heads, oph),
                        gat_ad[i].astype(f32))
        nlh = heads // 2
        banks = []
        for cc in range(2):
            hs = cc * nlh
            ncol = max(nlh, 1)
            banks.append(jnp.concatenate(
                [_pad16(ws[:, hs:hs + ncol]), _pad16(wd[:, hs:hs + ncol])],
                axis=1))
        wsd = jnp.stack(banks, 0)
        tab, al = _k_dense(h, wt, wsd, fh)
        acc, den = _sc_edge(tab, al, srcoff, dstoff, dst3, fh, heads)
        ea = _expand_mats(heads, oph, fh)
        bias = gat_b[i].astype(f32).reshape(1, -1)
        g = ln_g[i].astype(f32).reshape(1, -1)
        beta = ln_b[i].astype(f32).reshape(1, -1)
        if i < 3:
            h = _k_norm(acc, den, ea, bias, g, beta, fh, fo)
        else:
            emb = _k_emb(acc, den, ea, bias, g, beta, h0,
                         skip_W.astype(f32), skip_b.astype(f32).reshape(1, -1),
                         fh, fo)

    qf, qm = _k_q(emb[:nq], fa_W1.astype(f32), fa_b1.astype(f32).reshape(1, -1),
                  fa_W2.astype(f32), fa_b2.astype(f32).reshape(1, -1),
                  mo_W1.astype(f32), mo_b1.astype(f32).reshape(1, -1),
                  mo_W2.astype(f32), mo_b2.astype(f32).reshape(1, -1),
                  bil_W.astype(f32), nq)

    gender_p = jnp.pad(gender.astype(f32), (0, _NP - _N), constant_values=5.0)
    genr_p = jnp.pad(generation.astype(f32), (0, _NP - _N))
    gender8 = jnp.broadcast_to(gender_p[None, :], (8, _NP))
    genr8 = jnp.broadcast_to(genr_p[None, :], (8, _NP))
    genq = generation.astype(f32)[:nq].reshape(nq, 1)
    bb = bil_b.astype(f32).reshape(1, 1)

    father, mother = _k_score(qf, qm, emb, genr8, gender8, genq, bb, nq)
    return father[:, :_N], mother[:, :_N], emb[:_N]
